# Initial kernel scaffold; baseline (speedup 1.0000x reference)
import jax
import jax.numpy as jnp
from jax import lax
from jax.experimental import pallas as pl
from jax.experimental.pallas import tpu as pltpu
from jax.experimental.pallas import tpu_sc as plsc

NC = 2
NS = 16
NW = NC * NS

E = 320000
D = 128
EPW = E // NW
H = 80
CH = 400
HPB = CH // H
NCH = EPW // CH
GPC = CH // 16


def _body(z_hbm, ei_hbm, out_hbm, idx_u, idx_v, rows_u, rows_v, out_v, sem):
    cid = lax.axis_index("c")
    sid = lax.axis_index("s")
    wid = sid * NC + cid
    row0 = wid * (EPW // H)

    def chunk_body(c, carry):
        r0 = row0 + c * HPB
        pltpu.sync_copy(ei_hbm.at[0, pl.ds(r0, HPB)], idx_u)
        pltpu.sync_copy(ei_hbm.at[1, pl.ds(r0, HPB)], idx_v)
        cps = []
        for j in range(HPB):
            cps.append(pltpu.async_copy(
                z_hbm.at[idx_u.at[j]], rows_u.at[pl.ds(j * H, H)], sem))
        for j in range(HPB):
            cps.append(pltpu.async_copy(
                z_hbm.at[idx_v.at[j]], rows_v.at[pl.ds(j * H, H)], sem))
        for cp in cps:
            cp.wait()

        def group_body(g, carry2):
            ridx = g * 16 + lax.iota(jnp.int32, 16)

            def f_body(f, acc):
                fv = jnp.full((16,), f, jnp.int32)
                uf = plsc.load_gather(rows_u, [ridx, fv])
                vf = plsc.load_gather(rows_v, [ridx, fv])
                return acc + uf * vf

            acc = lax.fori_loop(0, D, f_body, jnp.zeros((16,), jnp.float32),
                                unroll=8)
            out_v[pl.ds(g * 16, 16)] = 1.0 / (1.0 + jnp.exp(-acc))
            return carry2

        lax.fori_loop(0, GPC, group_body, 0)
        pltpu.sync_copy(out_v, out_hbm.at[pl.ds(wid * EPW + c * CH, CH)])
        return carry

    lax.fori_loop(0, NCH, chunk_body, 0)


@jax.jit
def kernel(z, edge_index):
    ei = edge_index.astype(jnp.int32).reshape(2, E // H, H)
    mesh = plsc.VectorSubcoreMesh(core_axis_name="c", subcore_axis_name="s")
    return pl.kernel(
        _body,
        out_type=jax.ShapeDtypeStruct((E,), jnp.float32),
        mesh=mesh,
        scratch_types=[
            pltpu.VMEM((HPB, H), jnp.int32),
            pltpu.VMEM((HPB, H), jnp.int32),
            pltpu.VMEM((CH, D), jnp.float32),
            pltpu.VMEM((CH, D), jnp.float32),
            pltpu.VMEM((CH,), jnp.float32),
            pltpu.SemaphoreType.DMA,
        ],
    )(z, ei)


# speedup vs baseline: 1.2005x; 1.2005x over previous
import jax
import jax.numpy as jnp
from jax import lax
from jax.experimental import pallas as pl
from jax.experimental.pallas import tpu as pltpu
from jax.experimental.pallas import tpu_sc as plsc

NC = 2
NS = 16
NW = NC * NS

E = 320000
D = 128
EPW = E // NW
H = 80
CH = 400
HPB = CH // H
NCH = EPW // CH
GPC = CH // 16


def _body(z_hbm, src_hbm, dst_hbm, out_hbm, idx_u, idx_v, rows_u, rows_v,
          out_v, sem):
    cid = lax.axis_index("c")
    sid = lax.axis_index("s")
    wid = sid * NC + cid
    e0 = wid * EPW

    def chunk_body(c, carry):
        base = e0 + c * CH
        pltpu.sync_copy(src_hbm.at[pl.ds(base, CH)], idx_u)
        pltpu.sync_copy(dst_hbm.at[pl.ds(base, CH)], idx_v)
        cps = []
        for j in range(HPB):
            cps.append(pltpu.async_copy(
                z_hbm.at[idx_u.at[pl.ds(j * H, H)]],
                rows_u.at[pl.ds(j * H, H)], sem))
        for j in range(HPB):
            cps.append(pltpu.async_copy(
                z_hbm.at[idx_v.at[pl.ds(j * H, H)]],
                rows_v.at[pl.ds(j * H, H)], sem))
        for cp in cps:
            cp.wait()

        def group_body(g, carry2):
            ridx = g * 16 + lax.iota(jnp.int32, 16)

            def f_body(f, acc):
                fv = jnp.full((16,), f, jnp.int32)
                uf = plsc.load_gather(rows_u, [ridx, fv])
                vf = plsc.load_gather(rows_v, [ridx, fv])
                return acc + uf * vf

            acc = lax.fori_loop(0, D, f_body, jnp.zeros((16,), jnp.float32),
                                unroll=8)
            out_v[pl.ds(g * 16, 16)] = 1.0 / (1.0 + jnp.exp(-acc))
            return carry2

        lax.fori_loop(0, GPC, group_body, 0)
        pltpu.sync_copy(out_v, out_hbm.at[pl.ds(base, CH)])
        return carry

    lax.fori_loop(0, NCH, chunk_body, 0)


@jax.jit
def kernel(z, edge_index):
    ei = edge_index.astype(jnp.int32)
    mesh = plsc.VectorSubcoreMesh(core_axis_name="c", subcore_axis_name="s")
    return pl.kernel(
        _body,
        out_type=jax.ShapeDtypeStruct((E,), jnp.float32),
        mesh=mesh,
        compiler_params=pltpu.CompilerParams(needs_layout_passes=False),
        scratch_types=[
            pltpu.VMEM((CH,), jnp.int32),
            pltpu.VMEM((CH,), jnp.int32),
            pltpu.VMEM((CH, D), jnp.float32),
            pltpu.VMEM((CH, D), jnp.float32),
            pltpu.VMEM((CH,), jnp.float32),
            pltpu.SemaphoreType.DMA,
        ],
    )(z, ei[0], ei[1])


# speedup vs baseline: 3.2507x; 2.7077x over previous
import jax
import jax.numpy as jnp
from jax import lax
from jax.experimental import pallas as pl
from jax.experimental.pallas import tpu as pltpu
from jax.experimental.pallas import tpu_sc as plsc

NC = 2
NS = 16
NW = NC * NS

E = 320000
D = 128
EPW = E // NW
H = 80
CH = 400
HPB = CH // H
NCH = EPW // CH
GPC = CH // 16


def _body(z_hbm, src_hbm, dst_hbm, out_hbm, idx_u, idx_v, rows_u, rows_v,
          out_v, sem):
    cid = lax.axis_index("c")
    sid = lax.axis_index("s")
    wid = sid * NC + cid
    e0 = wid * EPW

    def chunk_body(c, carry):
        base = e0 + c * CH
        pltpu.sync_copy(src_hbm.at[pl.ds(base, CH)], idx_u)
        pltpu.sync_copy(dst_hbm.at[pl.ds(base, CH)], idx_v)
        cps = []
        for j in range(HPB):
            cps.append(pltpu.async_copy(
                z_hbm.at[idx_u.at[pl.ds(j * H, H)]],
                rows_u.at[pl.ds(j * H, H)], sem))
        for j in range(HPB):
            cps.append(pltpu.async_copy(
                z_hbm.at[idx_v.at[pl.ds(j * H, H)]],
                rows_v.at[pl.ds(j * H, H)], sem))
        for cp in cps:
            cp.wait()

        lane = lax.iota(jnp.int32, 16)
        rots = [(lane + s) % 16 for s in (1, 2, 4, 8)]

        def rot(x, perm):
            return lax.gather(
                x, perm[:, None],
                lax.GatherDimensionNumbers(
                    offset_dims=(), collapsed_slice_dims=(0,),
                    start_index_map=(0,)),
                (1,), mode=lax.GatherScatterMode.PROMISE_IN_BOUNDS)

        def group_body(g, carry2):
            acc = jnp.zeros((16,), jnp.float32)
            for j in range(16):
                e = g * 16 + j
                p = None
                for k in range(D // 16):
                    u = rows_u[e, pl.ds(k * 16, 16)]
                    v = rows_v[e, pl.ds(k * 16, 16)]
                    t = u * v
                    p = t if p is None else p + t
                for perm in rots:
                    p = p + rot(p, perm)
                acc = jnp.where(lane == j, p, acc)
            out_v[pl.ds(g * 16, 16)] = 1.0 / (1.0 + jnp.exp(-acc))
            return carry2

        lax.fori_loop(0, GPC, group_body, 0)
        pltpu.sync_copy(out_v, out_hbm.at[pl.ds(base, CH)])
        return carry

    lax.fori_loop(0, NCH, chunk_body, 0)


@jax.jit
def kernel(z, edge_index):
    ei = edge_index.astype(jnp.int32)
    mesh = plsc.VectorSubcoreMesh(core_axis_name="c", subcore_axis_name="s")
    return pl.kernel(
        _body,
        out_type=jax.ShapeDtypeStruct((E,), jnp.float32),
        mesh=mesh,
        compiler_params=pltpu.CompilerParams(needs_layout_passes=False),
        scratch_types=[
            pltpu.VMEM((CH,), jnp.int32),
            pltpu.VMEM((CH,), jnp.int32),
            pltpu.VMEM((CH, D), jnp.float32),
            pltpu.VMEM((CH, D), jnp.float32),
            pltpu.VMEM((CH,), jnp.float32),
            pltpu.SemaphoreType.DMA,
        ],
    )(z, ei[0], ei[1])


# speedup vs baseline: 8.8094x; 2.7100x over previous
import jax
import jax.numpy as jnp
from jax import lax
from jax.experimental import pallas as pl
from jax.experimental.pallas import tpu as pltpu
from jax.experimental.pallas import tpu_sc as plsc

NC = 2
NS = 16
NW = NC * NS

E = 320000
D = 128
EPW = E // NW
CH = 80
NCH = EPW // CH
GPC = CH // 16


def _body(z_hbm, src_hbm, dst_hbm, out_hbm, idx_u, idx_v, rows_u, rows_v,
          out_v, sem):
    cid = lax.axis_index("c")
    sid = lax.axis_index("s")
    wid = sid * NC + cid
    e0 = wid * EPW

    pltpu.sync_copy(src_hbm.at[pl.ds(e0, EPW)], idx_u)
    pltpu.sync_copy(dst_hbm.at[pl.ds(e0, EPW)], idx_v)

    def fire(c, b):
        pltpu.async_copy(z_hbm.at[idx_u.at[pl.ds(c * CH, CH)]],
                         rows_u.at[b], sem.at[b])
        pltpu.async_copy(z_hbm.at[idx_v.at[pl.ds(c * CH, CH)]],
                         rows_v.at[b], sem.at[b])

    def drain(c, b):
        pltpu.make_async_copy(z_hbm.at[idx_u.at[pl.ds(c * CH, CH)]],
                              rows_u.at[b], sem.at[b]).wait()
        pltpu.make_async_copy(z_hbm.at[idx_v.at[pl.ds(c * CH, CH)]],
                              rows_v.at[b], sem.at[b]).wait()

    fire(0, 0)

    lane = lax.iota(jnp.int32, 16)
    rots = [(lane + s) % 16 for s in (1, 2, 4, 8)]

    def rot(x, perm):
        return lax.gather(
            x, perm[:, None],
            lax.GatherDimensionNumbers(
                offset_dims=(), collapsed_slice_dims=(0,),
                start_index_map=(0,)),
            (1,), mode=lax.GatherScatterMode.PROMISE_IN_BOUNDS)

    def chunk_body(c, carry):
        b = c % 2
        nb = (c + 1) % 2

        @pl.when(c + 1 < NCH)
        def _():
            fire(c + 1, nb)

        drain(c, b)

        def group_body(g, carry2):
            def edge_body(j, acc):
                e = g * 16 + j
                p = None
                for k in range(D // 16):
                    u = rows_u[b, e, pl.ds(k * 16, 16)]
                    v = rows_v[b, e, pl.ds(k * 16, 16)]
                    t = u * v
                    p = t if p is None else p + t
                for perm in rots:
                    p = p + rot(p, perm)
                return jnp.where(lane == j, p, acc)

            acc = lax.fori_loop(0, 16, edge_body,
                                jnp.zeros((16,), jnp.float32), unroll=4)
            out_v[pl.ds(c * CH + g * 16, 16)] = 1.0 / (1.0 + jnp.exp(-acc))
            return carry2

        lax.fori_loop(0, GPC, group_body, 0)
        return carry

    lax.fori_loop(0, NCH, chunk_body, 0)
    pltpu.sync_copy(out_v, out_hbm.at[pl.ds(e0, EPW)])


@jax.jit
def kernel(z, edge_index):
    ei = edge_index.astype(jnp.int32)
    mesh = plsc.VectorSubcoreMesh(core_axis_name="c", subcore_axis_name="s")
    return pl.kernel(
        _body,
        out_type=jax.ShapeDtypeStruct((E,), jnp.float32),
        mesh=mesh,
        compiler_params=pltpu.CompilerParams(needs_layout_passes=False),
        scratch_types=[
            pltpu.VMEM((EPW,), jnp.int32),
            pltpu.VMEM((EPW,), jnp.int32),
            pltpu.VMEM((2, CH, D), jnp.float32),
            pltpu.VMEM((2, CH, D), jnp.float32),
            pltpu.VMEM((EPW,), jnp.float32),
            pltpu.SemaphoreType.DMA((2,)),
        ],
    )(z, ei[0], ei[1])
